# baseline (speedup 1.0000x reference)
import jax
import jax.numpy as jnp
from jax import lax
from jax.experimental import pallas as pl
from jax.experimental.pallas import tpu as pltpu
from jax.experimental.pallas import tpu_sc as plsc

VOCAB = 1000100
DIM = 64
LANES = 16
NUM_CORES = 2
NUM_SUBCORES = 16
NW = NUM_CORES * NUM_SUBCORES

NBUF = 10
LAG = 5


def _body(ids_hbm, w_hbm, out_hbm, idx_v, rows, gsems, osems,
          *, rows_per_w, hist):
    rows = list(rows)
    gsems = list(gsems)
    osems = list(osems)
    wid = lax.axis_index("s") * NUM_CORES + lax.axis_index("c")
    base = wid * rows_per_w
    hsz = 104
    hoff = hist - hsz
    assert hoff % 8 == 0 and hoff < hsz
    units = rows_per_w * 2

    pltpu.sync_copy(ids_hbm.at[pl.ds(base, rows_per_w)], idx_v)

    offs = list(range(0, hist - LANES + 1, LANES))
    if offs[-1] + LANES < hist:
        offs.append(hist - LANES)

    def clamp_row(r, _):
        for o in offs:
            sl = pl.ds(o, LANES)
            idx_v[r, sl] = jnp.minimum(idx_v[r, sl], VOCAB - 1)
        return 0

    lax.fori_loop(0, rows_per_w, clamp_row, 0, unroll=False)

    def start_gather(u, b):
        r = u >> 1
        off = pl.multiple_of((u & 1) * hoff, 8)
        pltpu.async_copy(w_hbm.at[idx_v.at[r, pl.ds(off, hsz)]],
                         rows[b], gsems[b])

    def wait_gather(b):
        pltpu.make_async_copy(out_hbm.at[0, pl.ds(0, hsz)], rows[b],
                              gsems[b]).wait()

    def start_out(u, b):
        r = u >> 1
        off = pl.multiple_of((u & 1) * hoff, 8)
        pltpu.async_copy(rows[b], out_hbm.at[base + r, pl.ds(off, hsz)],
                         osems[b])

    def wait_out(b):
        pltpu.make_async_copy(rows[b], out_hbm.at[0, pl.ds(0, hsz)],
                              osems[b]).wait()

    n_groups = (units + LAG) // NBUF + 1

    def group(g, _):
        for b in range(NBUF):
            u = g * NBUF + b

            @pl.when(u < units)
            def _gather():
                @pl.when(u >= NBUF)
                def _drain():
                    wait_out(b)

                start_gather(u, b)

            j = u - LAG
            bj = (b + NBUF - LAG) % NBUF

            @pl.when(jnp.logical_and(j >= 0, j < units))
            def _out():
                wait_gather(bj)
                start_out(j, bj)

        return 0

    lax.fori_loop(0, n_groups, group, 0, unroll=False)

    for b in range(NBUF):
        wait_out(b)


def kernel(input_ids, weight):
    batch, hist = input_ids.shape
    assert batch % NW == 0
    assert hist % 2 == 0
    rows_per_w = batch // NW

    ids = input_ids.astype(jnp.int32)

    mesh = plsc.VectorSubcoreMesh(
        core_axis_name="c", subcore_axis_name="s",
        num_cores=NUM_CORES, num_subcores=NUM_SUBCORES)

    scratch = (
        [pltpu.VMEM((rows_per_w, hist), jnp.int32)]
        + [pltpu.VMEM((104, DIM), jnp.float32) for _ in range(NBUF)]
        + [pltpu.SemaphoreType.DMA for _ in range(2 * NBUF)]
    )

    def body(ids_hbm, w_hbm, out_hbm, *scr):
        _body(ids_hbm, w_hbm, out_hbm,
              scr[0], scr[1:1 + NBUF], scr[1 + NBUF:1 + 2 * NBUF],
              scr[1 + 2 * NBUF:], rows_per_w=rows_per_w, hist=hist)

    return pl.kernel(
        body,
        out_type=jax.ShapeDtypeStruct((batch, hist, DIM), jnp.float32),
        mesh=mesh,
        scratch_types=scratch,
        compiler_params=pltpu.CompilerParams(use_tc_tiling_on_sc=False),
    )(ids, weight)


# speedup vs baseline: 1.0063x; 1.0063x over previous
import jax
import jax.numpy as jnp
from jax import lax
from jax.experimental import pallas as pl
from jax.experimental.pallas import tpu as pltpu
from jax.experimental.pallas import tpu_sc as plsc

VOCAB = 1000100
DIM = 64
LANES = 16
NUM_CORES = 2
NUM_SUBCORES = 16
NW = NUM_CORES * NUM_SUBCORES

NBUF = 4
LAG = 2


def _body(ids_hbm, w_hbm, out_hbm, idx_v, rows, gsems, osems,
          *, rows_per_w, hist):
    rows = list(rows)
    gsems = list(gsems)
    osems = list(osems)
    wid = lax.axis_index("s") * NUM_CORES + lax.axis_index("c")
    base = wid * rows_per_w

    pltpu.sync_copy(ids_hbm.at[pl.ds(base, rows_per_w)], idx_v)

    offs = list(range(0, hist - LANES + 1, LANES))
    if offs[-1] + LANES < hist:
        offs.append(hist - LANES)

    def clamp_row(r, _):
        for o in offs:
            sl = pl.ds(o, LANES)
            idx_v[r, sl] = jnp.minimum(idx_v[r, sl], VOCAB - 1)
        return 0

    lax.fori_loop(0, rows_per_w, clamp_row, 0, unroll=False)

    split = 128
    rest = hist - split

    def start_gather(r, b):
        pltpu.async_copy(w_hbm.at[idx_v.at[r, pl.ds(0, split)]],
                         rows[b].at[pl.ds(0, split)], gsems[b])
        pltpu.async_copy(w_hbm.at[idx_v.at[r, pl.ds(split, rest)]],
                         rows[b].at[pl.ds(split, rest)], gsems[b])

    def wait_gather(b):
        pltpu.make_async_copy(out_hbm.at[0], rows[b], gsems[b]).wait()

    def start_out(r, b):
        pltpu.async_copy(rows[b], out_hbm.at[base + r], osems[b])

    def wait_out(b):
        pltpu.make_async_copy(rows[b], out_hbm.at[0], osems[b]).wait()

    n_groups = (rows_per_w + LAG) // NBUF + 1

    def group(g, _):
        for b in range(NBUF):
            r = g * NBUF + b

            @pl.when(r < rows_per_w)
            def _gather():
                @pl.when(r >= NBUF)
                def _drain():
                    wait_out(b)

                start_gather(r, b)

            j = r - LAG
            bj = (b + NBUF - LAG) % NBUF

            @pl.when(jnp.logical_and(j >= 0, j < rows_per_w))
            def _out():
                wait_gather(bj)
                start_out(j, bj)

        return 0

    lax.fori_loop(0, n_groups, group, 0, unroll=False)

    for b in range(NBUF):
        wait_out(b)


def kernel(input_ids, weight):
    batch, hist = input_ids.shape
    assert batch % NW == 0
    rows_per_w = batch // NW

    ids = input_ids.astype(jnp.int32)

    mesh = plsc.VectorSubcoreMesh(
        core_axis_name="c", subcore_axis_name="s",
        num_cores=NUM_CORES, num_subcores=NUM_SUBCORES)

    scratch = (
        [pltpu.VMEM((rows_per_w, hist), jnp.int32)]
        + [pltpu.VMEM((hist, DIM), jnp.float32) for _ in range(NBUF)]
        + [pltpu.SemaphoreType.DMA for _ in range(2 * NBUF)]
    )

    def body(ids_hbm, w_hbm, out_hbm, *scr):
        _body(ids_hbm, w_hbm, out_hbm,
              scr[0], scr[1:1 + NBUF], scr[1 + NBUF:1 + 2 * NBUF],
              scr[1 + 2 * NBUF:], rows_per_w=rows_per_w, hist=hist)

    return pl.kernel(
        body,
        out_type=jax.ShapeDtypeStruct((batch, hist, DIM), jnp.float32),
        mesh=mesh,
        scratch_types=scratch,
        compiler_params=pltpu.CompilerParams(use_tc_tiling_on_sc=False),
    )(ids, weight)
